# Initial kernel scaffold; baseline (speedup 1.0000x reference)
import jax
import jax.numpy as jnp
from jax.experimental import pallas as pl


def kernel(x, edge_index, edge_attr, batch, W_rel0, b_rel0, W_root0, W_rel1, b_rel1, W_root1, W_rel2, b_rel2, W_root2, W_lin, b_lin):
    raise NotImplementedError("write your pallas kernel here")


# speedup vs baseline: 7.1879x; 7.1879x over previous
import functools

import jax
import jax.numpy as jnp
from jax import lax
from jax.experimental import pallas as pl
from jax.experimental.pallas import tpu as pltpu
from jax.experimental.pallas import tpu_sc as plsc

NN = 50000
EE = 800000
HH = 64
GG = 64

BLK = 512
NPAD = 50176
NBLKS = NPAD // BLK
ROWS_PER_TILE = NPAD // 16
EPAD = 802816
NGROUPS = EPAD // 128

_MESH = dict(core_axis_name="c", subcore_axis_name="s")


def _sc_seg1_body(x_hbm, src_hbm, dst_hbm, ew_hbm, out_hbm,
                  src_v, dst_v, ew_v, rows_v, zero_v, acc_sh, gsem, ssem):
    c = lax.axis_index("c")
    s = lax.axis_index("s")
    for i in range(ROWS_PER_TILE // 16):
        zero_v[pl.ds(i * 16, 16)] = jnp.zeros((16,), jnp.float32)
    pltpu.sync_copy(zero_v, acc_sh.at[pl.ds(s * ROWS_PER_TILE, ROWS_PER_TILE)])
    plsc.subcore_barrier()

    w = s * 2 + c
    g0 = w * (NGROUPS // 32)

    def chunk_body(k, carry):
        gg = g0 + k * 4
        pltpu.sync_copy(src_hbm.at[pl.ds(gg, 4)], src_v)
        pltpu.sync_copy(dst_hbm.at[pl.ds(gg, 4)], dst_v)
        pltpu.sync_copy(ew_hbm.at[pl.ds(gg * 128, 512)], ew_v)
        cps = [pltpu.async_copy(x_hbm.at[src_v.at[j]],
                                rows_v.at[pl.ds(j * 128, 128)], gsem)
               for j in range(4)]
        for cp in cps:
            cp.wait()
        for t in range(32):
            sl = pl.ds(t * 16, 16)
            rows_v[sl] = rows_v[sl] * ew_v[sl]
        adds = [pltpu.async_copy(rows_v.at[pl.ds(j * 128, 128)],
                                 acc_sh.at[dst_v.at[j]], ssem, add=True)
                for j in range(4)]
        for cp in adds:
            cp.wait()
        return carry

    lax.fori_loop(0, (NGROUPS // 32) // 4, chunk_body, 0)
    plsc.subcore_barrier()
    pltpu.sync_copy(acc_sh.at[pl.ds(s * ROWS_PER_TILE, ROWS_PER_TILE)],
                    zero_v)
    pltpu.sync_copy(
        zero_v,
        out_hbm.at[pl.ds(c * NPAD + s * ROWS_PER_TILE, ROWS_PER_TILE)])


def _sc_seg1(x_p, src2d, dst2d, ew1d):
    return pl.kernel(
        _sc_seg1_body,
        out_type=jax.ShapeDtypeStruct((2 * NPAD,), jnp.float32),
        mesh=plsc.VectorSubcoreMesh(**_MESH),
        scratch_types=[
            pltpu.VMEM((4, 128), jnp.int32),
            pltpu.VMEM((4, 128), jnp.int32),
            pltpu.VMEM((512,), jnp.float32),
            pltpu.VMEM((512,), jnp.float32),
            pltpu.VMEM((ROWS_PER_TILE,), jnp.float32),
            pltpu.VMEM_SHARED((NPAD,), jnp.float32),
            pltpu.SemaphoreType.DMA,
            pltpu.SemaphoreType.DMA,
        ],
    )(x_p, src2d, dst2d, ew1d)


def _sc_seg16_body(tab_hbm, src_hbm, dst_hbm, ew_hbm, out_hbm,
                   src_v, dst_v, ew_v, rows_v, zero_v, acc_sh, gsem, ssem):
    c = lax.axis_index("c")
    s = lax.axis_index("s")
    for r in range(64):
        zero_v[r, pl.ds(0, 16)] = jnp.zeros((16,), jnp.float32)
    gpt = NGROUPS // 16

    for p in range(2):
        q = 2 * p + c
        for i in range(ROWS_PER_TILE // 64):
            pltpu.sync_copy(zero_v,
                            acc_sh.at[pl.ds(s * ROWS_PER_TILE + i * 64, 64)])
        plsc.subcore_barrier()

        def chunk_body(k, carry):
            gg = s * gpt + k * 8
            pltpu.sync_copy(src_hbm.at[pl.ds(gg, 8)], src_v)
            pltpu.sync_copy(dst_hbm.at[pl.ds(gg, 8)], dst_v)
            pltpu.sync_copy(ew_hbm.at[pl.ds(gg * 128, 1024)], ew_v)
            cps = [pltpu.async_copy(tab_hbm.at[q].at[src_v.at[j]],
                                    rows_v.at[pl.ds(j * 128, 128)], gsem)
                   for j in range(8)]
            for cp in cps:
                cp.wait()

            def scale_body(t, cc):
                ew16 = ew_v[pl.ds(t * 16, 16)]
                base = t * 16
                for u in range(16):
                    e = base + u
                    rows_v[e, pl.ds(0, 16)] = (rows_v[e, pl.ds(0, 16)]
                                               * ew16[u])
                return cc

            lax.fori_loop(0, 64, scale_body, 0)
            adds = [pltpu.async_copy(rows_v.at[pl.ds(j * 128, 128)],
                                     acc_sh.at[dst_v.at[j]], ssem, add=True)
                    for j in range(8)]
            for cp in adds:
                cp.wait()
            return carry

        lax.fori_loop(0, gpt // 8, chunk_body, 0)
        plsc.subcore_barrier()
        base = s * ROWS_PER_TILE
        for i in range(3):
            pltpu.sync_copy(acc_sh.at[pl.ds(base + i * 1024, 1024)], rows_v)
            pltpu.sync_copy(rows_v,
                            out_hbm.at[q, pl.ds(base + i * 1024, 1024)])
        pltpu.sync_copy(acc_sh.at[pl.ds(base + 3072, 64)],
                        rows_v.at[pl.ds(0, 64)])
        pltpu.sync_copy(rows_v.at[pl.ds(0, 64)],
                        out_hbm.at[q, pl.ds(base + 3072, 64)])
        plsc.subcore_barrier()


def _sc_seg16(tab, src2d, dst2d, ew1d):
    return pl.kernel(
        _sc_seg16_body,
        out_type=jax.ShapeDtypeStruct((4, NPAD, 16), jnp.float32),
        mesh=plsc.VectorSubcoreMesh(**_MESH),
        compiler_params=pltpu.CompilerParams(use_tc_tiling_on_sc=False),
        scratch_types=[
            pltpu.VMEM((8, 128), jnp.int32),
            pltpu.VMEM((8, 128), jnp.int32),
            pltpu.VMEM((1024,), jnp.float32),
            pltpu.VMEM((1024, 16), jnp.float32),
            pltpu.VMEM((64, 16), jnp.float32),
            pltpu.VMEM_SHARED((NPAD, 16), jnp.float32),
            pltpu.SemaphoreType.DMA,
            pltpu.SemaphoreType.DMA,
        ],
    )(tab, src2d, dst2d, ew1d)


def _tc1_body(agg_ref, x_ref, wrel_ref, wroot_ref, b_ref, out_ref):
    a = agg_ref[0] + agg_ref[1]
    xv = x_ref[...]
    h = a * wrel_ref[...] + xv * wroot_ref[...] + b_ref[...]
    h = jnp.maximum(h, 0.0)
    for qq in range(4):
        out_ref[qq] = h[:, qq * 16:(qq + 1) * 16]


def _tc1(agg0, x_p, W_rel0, W_root0, b_rel0):
    return pl.pallas_call(
        _tc1_body,
        grid=(NBLKS,),
        in_specs=[
            pl.BlockSpec((2, BLK, 1), lambda i: (0, i, 0)),
            pl.BlockSpec((BLK, 1), lambda i: (i, 0)),
            pl.BlockSpec((1, HH), lambda i: (0, 0)),
            pl.BlockSpec((1, HH), lambda i: (0, 0)),
            pl.BlockSpec((1, HH), lambda i: (0, 0)),
        ],
        out_specs=pl.BlockSpec((4, BLK, 16), lambda i: (0, i, 0)),
        out_shape=jax.ShapeDtypeStruct((4, NPAD, 16), jnp.float32),
    )(agg0.reshape(2, NPAD, 1), x_p.reshape(NPAD, 1),
      W_rel0.reshape(1, HH), W_root0.reshape(1, HH), b_rel0.reshape(1, HH))


def _tc_mid_body(agg_ref, hp_ref, wrel_ref, wroot_ref, b_ref, out_ref):
    a = jnp.concatenate([agg_ref[qq] for qq in range(4)], axis=1)
    hp = jnp.concatenate([hp_ref[qq] for qq in range(4)], axis=1)
    h = (jnp.dot(a, wrel_ref[...], preferred_element_type=jnp.float32)
         + jnp.dot(hp, wroot_ref[...], preferred_element_type=jnp.float32)
         + b_ref[...])
    h = jnp.maximum(h, 0.0)
    for qq in range(4):
        out_ref[qq] = h[:, qq * 16:(qq + 1) * 16]


def _tc_mid(agg, h_prev, W_rel, W_root, b_rel):
    return pl.pallas_call(
        _tc_mid_body,
        grid=(NBLKS,),
        in_specs=[
            pl.BlockSpec((4, BLK, 16), lambda i: (0, i, 0)),
            pl.BlockSpec((4, BLK, 16), lambda i: (0, i, 0)),
            pl.BlockSpec((HH, HH), lambda i: (0, 0)),
            pl.BlockSpec((HH, HH), lambda i: (0, 0)),
            pl.BlockSpec((1, HH), lambda i: (0, 0)),
        ],
        out_specs=pl.BlockSpec((4, BLK, 16), lambda i: (0, i, 0)),
        out_shape=jax.ShapeDtypeStruct((4, NPAD, 16), jnp.float32),
    )(agg, h_prev, W_rel, W_root, b_rel.reshape(1, HH))


def _tc_final_body(agg_ref, hp_ref, wrel_ref, wroot_ref, b_ref, wl_ref,
                   bl_ref, batch_ref, out_ref, sums, counts):
    i = pl.program_id(0)

    @pl.when(i == 0)
    def _():
        sums[...] = jnp.zeros_like(sums)
        counts[...] = jnp.zeros_like(counts)

    a = jnp.concatenate([agg_ref[qq] for qq in range(4)], axis=1)
    hp = jnp.concatenate([hp_ref[qq] for qq in range(4)], axis=1)
    h = (jnp.dot(a, wrel_ref[...], preferred_element_type=jnp.float32)
         + jnp.dot(hp, wroot_ref[...], preferred_element_type=jnp.float32)
         + b_ref[...])
    h = jnp.maximum(h, 0.0)
    s = jnp.sum(h * wl_ref[...], axis=1, keepdims=True)
    onehot = (batch_ref[...] ==
              lax.broadcasted_iota(jnp.int32, (BLK, GG), 1)
              ).astype(jnp.float32)
    sums[...] += jnp.sum(onehot * s, axis=0, keepdims=True)
    counts[...] += jnp.sum(onehot, axis=0, keepdims=True)

    @pl.when(i == NBLKS - 1)
    def _():
        out_ref[...] = sums[...] / jnp.maximum(counts[...], 1.0) + bl_ref[...]


def _tc_final(agg, h_prev, W_rel, W_root, b_rel, W_lin, b_lin, batch_p):
    return pl.pallas_call(
        _tc_final_body,
        grid=(NBLKS,),
        in_specs=[
            pl.BlockSpec((4, BLK, 16), lambda i: (0, i, 0)),
            pl.BlockSpec((4, BLK, 16), lambda i: (0, i, 0)),
            pl.BlockSpec((HH, HH), lambda i: (0, 0)),
            pl.BlockSpec((HH, HH), lambda i: (0, 0)),
            pl.BlockSpec((1, HH), lambda i: (0, 0)),
            pl.BlockSpec((1, HH), lambda i: (0, 0)),
            pl.BlockSpec((1, 1), lambda i: (0, 0)),
            pl.BlockSpec((BLK, 1), lambda i: (i, 0)),
        ],
        out_specs=pl.BlockSpec((1, GG), lambda i: (0, 0)),
        out_shape=jax.ShapeDtypeStruct((1, GG), jnp.float32),
        scratch_shapes=[
            pltpu.VMEM((1, GG), jnp.float32),
            pltpu.VMEM((1, GG), jnp.float32),
        ],
    )(agg, h_prev, W_rel, W_root, b_rel.reshape(1, HH),
      W_lin.reshape(1, HH), b_lin.reshape(1, 1), batch_p.reshape(NPAD, 1))


def kernel(x, edge_index, edge_attr, batch,
           W_rel0, b_rel0, W_root0,
           W_rel1, b_rel1, W_root1,
           W_rel2, b_rel2, W_root2,
           W_lin, b_lin):
    src = edge_index[0]
    dst = edge_index[1]
    ew = edge_attr.reshape(-1)

    pad_e = EPAD - EE
    zi = jnp.zeros((pad_e,), jnp.int32)
    src2d = jnp.concatenate([src, zi]).reshape(NGROUPS, 128)
    dst2d = jnp.concatenate([dst, zi]).reshape(NGROUPS, 128)
    ew1d = jnp.concatenate([ew, jnp.zeros((pad_e,), jnp.float32)])

    x_p = jnp.concatenate([x.reshape(-1), jnp.zeros((NPAD - NN,), jnp.float32)])
    batch_p = jnp.concatenate(
        [batch, jnp.full((NPAD - NN,), GG, jnp.int32)])

    agg0 = _sc_seg1(x_p, src2d, dst2d, ew1d)
    h0 = _tc1(agg0, x_p, W_rel0, W_root0, b_rel0)
    agg1 = _sc_seg16(h0, src2d, dst2d, ew1d)
    h1 = _tc_mid(agg1, h0, W_rel1, W_root1, b_rel1)
    agg2 = _sc_seg16(h1, src2d, dst2d, ew1d)
    out = _tc_final(agg2, h1, W_rel2, W_root2, b_rel2, W_lin, b_lin, batch_p)
    return out.reshape(GG)


# speedup vs baseline: 12.2167x; 1.6996x over previous
import jax
import jax.numpy as jnp
from jax import lax
from jax.experimental import pallas as pl
from jax.experimental.pallas import tpu as pltpu
from jax.experimental.pallas import tpu_sc as plsc

NN = 50000
EE = 800000
HH = 64
GG = 64

BLK = 1024
NPAD = 50176
NBLKS = NPAD // BLK
ROWS_PER_TILE = NPAD // 16
EPAD = 802816
NGROUPS = EPAD // 128

_MESH = dict(core_axis_name="c", subcore_axis_name="s")


_SUB0 = 7
_PH0 = 2 * _SUB0
_IT0 = 2 * _PH0
_NIT0 = (NGROUPS // 32) // _IT0


def _sc_seg1_body(x_hbm, src_hbm, dst_hbm, ew_hbm, out_hbm,
                  srcP, dstP, ewP, srcQ, dstQ, ewQ,
                  rows0, rows1, zero_v, acc_sh,
                  g0sem, g1sem, sPsem, sQsem, csem):
    c = lax.axis_index("c")
    s = lax.axis_index("s")
    rows = [rows0, rows1]
    gsems = [g0sem, g1sem]
    for i in range(ROWS_PER_TILE // 16):
        zero_v[pl.ds(i * 16, 16)] = jnp.zeros((16,), jnp.float32)
    pltpu.sync_copy(zero_v, acc_sh.at[pl.ds(s * ROWS_PER_TILE,
                                            ROWS_PER_TILE)])
    plsc.subcore_barrier()

    w = s * 2 + c
    base_g = w * (NGROUPS // 32)

    def scale(rows_i, ew_ref, i):
        def scale_body(g, cc):
            sl = pl.ds(g * 16, 16)
            rows_i[sl] = rows_i[sl] * ew_ref[pl.ds(i * 896 + g * 16, 16)]
            return cc
        lax.fori_loop(0, _SUB0 * 8, scale_body, 0)

    def issue_gathers(src_ref, i):
        for j in range(_SUB0):
            pltpu.async_copy(x_hbm.at[src_ref.at[i * _SUB0 + j]],
                             rows[i].at[pl.ds(j * 128, 128)], gsems[i])

    def drain_streams(sem, n):
        for _ in range(n):
            pltpu.make_async_copy(x_hbm.at[pl.ds(0, 128)],
                                  rows0.at[pl.ds(0, 128)], sem).wait()

    def issue_scatters(dst_ref, i, sem):
        for j in range(_SUB0):
            pltpu.async_copy(rows[i].at[pl.ds(j * 128, 128)],
                             acc_sh.at[dst_ref.at[i * _SUB0 + j]], sem,
                             add=True)

    def do_phase(gg, src_ref, dst_ref, ew_ref, my_sem):
        cps = [pltpu.async_copy(src_hbm.at[pl.ds(gg, _PH0)], src_ref, csem),
               pltpu.async_copy(dst_hbm.at[pl.ds(gg, _PH0)], dst_ref, csem),
               pltpu.async_copy(ew_hbm.at[pl.ds(gg * 128, _PH0 * 128)],
                                ew_ref, csem)]
        for cp in cps:
            cp.wait()
        for i in range(2):
            issue_gathers(src_ref, i)
        for i in range(2):
            drain_streams(gsems[i], _SUB0)
            scale(rows[i], ew_ref, i)
            issue_scatters(dst_ref, i, my_sem)

    def iter_body(t, carry):
        gg = base_g + t * _IT0

        @pl.when(t > 0)
        def _():
            drain_streams(sQsem, _PH0)
        do_phase(gg, srcP, dstP, ewP, sPsem)
        drain_streams(sPsem, _PH0)
        do_phase(gg + _PH0, srcQ, dstQ, ewQ, sQsem)
        return carry

    lax.fori_loop(0, _NIT0, iter_body, 0)
    drain_streams(sQsem, _PH0)
    plsc.subcore_barrier()
    pltpu.sync_copy(acc_sh.at[pl.ds(s * ROWS_PER_TILE, ROWS_PER_TILE)],
                    zero_v)
    pltpu.sync_copy(
        zero_v,
        out_hbm.at[pl.ds(c * NPAD + s * ROWS_PER_TILE, ROWS_PER_TILE)])


def _sc_seg1(x_p, src2d, dst2d, ew1d):
    return pl.kernel(
        _sc_seg1_body,
        out_type=jax.ShapeDtypeStruct((2 * NPAD,), jnp.float32),
        mesh=plsc.VectorSubcoreMesh(**_MESH),
        compiler_params=pltpu.CompilerParams(use_tc_tiling_on_sc=False),
        scratch_types=[
            pltpu.VMEM((_PH0, 128), jnp.int32),
            pltpu.VMEM((_PH0, 128), jnp.int32),
            pltpu.VMEM((_PH0 * 128,), jnp.float32),
            pltpu.VMEM((_PH0, 128), jnp.int32),
            pltpu.VMEM((_PH0, 128), jnp.int32),
            pltpu.VMEM((_PH0 * 128,), jnp.float32),
            pltpu.VMEM((_SUB0 * 128,), jnp.float32),
            pltpu.VMEM((_SUB0 * 128,), jnp.float32),
            pltpu.VMEM((ROWS_PER_TILE,), jnp.float32),
            pltpu.VMEM_SHARED((NPAD,), jnp.float32),
            pltpu.SemaphoreType.DMA,
            pltpu.SemaphoreType.DMA,
            pltpu.SemaphoreType.DMA,
            pltpu.SemaphoreType.DMA,
            pltpu.SemaphoreType.DMA,
        ],
    )(x_p, src2d, dst2d, ew1d)


_SUB = 7
_NSUB = 4
_PHG = _SUB * _NSUB
_ITG = 2 * _PHG
_NIT = (NGROUPS // 16) // _ITG


def _sc_seg16_body(tab_hbm, src_hbm, dst_hbm, ew_hbm, out_hbm,
                   srcP, dstP, ewP, srcQ, dstQ, ewQ,
                   rows0, rows1, rows2, rows3, zero_v, acc_sh,
                   g0sem, g1sem, g2sem, g3sem, sPsem, sQsem, csem):
    c = lax.axis_index("c")
    s = lax.axis_index("s")
    rows = [rows0, rows1, rows2, rows3]
    gsems = [g0sem, g1sem, g2sem, g3sem]
    for r in range(64):
        zero_v[r, pl.ds(0, 16)] = jnp.zeros((16,), jnp.float32)

    def scale(rows_i, ew_ref, i):
        def scale_body(g, cc):
            ew16 = ew_ref[pl.ds(i * (_SUB * 128) + g * 16, 16)]
            base = g * 16
            for u in range(16):
                e = base + u
                rows_i[e, pl.ds(0, 16)] = rows_i[e, pl.ds(0, 16)] * ew16[u]
            return cc
        lax.fori_loop(0, _SUB * 8, scale_body, 0)

    for p in range(2):
        q = 2 * p + c
        qv = jnp.full((16,), 0, jnp.int32) + q

        def issue_gathers(src_ref, i):
            for j in range(_SUB):
                pltpu.async_copy(tab_hbm.at[src_ref.at[i * _SUB + j]],
                                 rows[i].at[pl.ds(j * 128, 128)], gsems[i])

        def drain_streams(sem, n):
            for _ in range(n):
                pltpu.make_async_copy(tab_hbm.at[pl.ds(0, 128)],
                                      rows0.at[pl.ds(0, 128)], sem).wait()

        def issue_scatters(dst_ref, i, sem):
            for j in range(_SUB):
                pltpu.async_copy(rows[i].at[pl.ds(j * 128, 128)],
                                 acc_sh.at[dst_ref.at[i * _SUB + j]], sem,
                                 add=True)

        def do_phase(gg, src_ref, dst_ref, ew_ref, my_sem):
            cps = [
                pltpu.async_copy(src_hbm.at[pl.ds(gg, _PHG)], src_ref, csem),
                pltpu.async_copy(dst_hbm.at[pl.ds(gg, _PHG)], dst_ref, csem),
                pltpu.async_copy(ew_hbm.at[pl.ds(gg * 128, _PHG * 128)],
                                 ew_ref, csem)]
            for cp in cps:
                cp.wait()

            def addq_body(g, cc):
                for k2 in range(8):
                    sl = pl.ds(k2 * 16, 16)
                    src_ref[g, sl] = src_ref[g, sl] + qv
                return cc
            lax.fori_loop(0, _PHG, addq_body, 0)
            for i in range(_NSUB):
                issue_gathers(src_ref, i)
            for i in range(_NSUB):
                drain_streams(gsems[i], _SUB)
                scale(rows[i], ew_ref, i)
                issue_scatters(dst_ref, i, my_sem)

        for i in range(ROWS_PER_TILE // 64):
            pltpu.async_copy(
                zero_v, acc_sh.at[pl.ds(s * ROWS_PER_TILE + i * 64, 64)],
                csem)
        for i in range(ROWS_PER_TILE // 64):
            pltpu.make_async_copy(
                zero_v, acc_sh.at[pl.ds(s * ROWS_PER_TILE, 64)], csem).wait()
        plsc.subcore_barrier()

        def iter_body(t, carry):
            gg = s * (NGROUPS // 16) + t * _ITG

            @pl.when(t > 0)
            def _():
                drain_streams(sQsem, _PHG)
            do_phase(gg, srcP, dstP, ewP, sPsem)
            drain_streams(sPsem, _PHG)
            do_phase(gg + _PHG, srcQ, dstQ, ewQ, sQsem)
            return carry

        lax.fori_loop(0, _NIT, iter_body, 0)
        drain_streams(sQsem, _PHG)
        plsc.subcore_barrier()
        base = s * ROWS_PER_TILE
        for i in range(3):
            pltpu.sync_copy(acc_sh.at[pl.ds(base + i * 896, 896)], rows0)
            pltpu.sync_copy(rows0, out_hbm.at[q, pl.ds(base + i * 896, 896)])
        pltpu.sync_copy(acc_sh.at[pl.ds(base + 2688, 448)],
                        rows0.at[pl.ds(0, 448)])
        pltpu.sync_copy(rows0.at[pl.ds(0, 448)],
                        out_hbm.at[q, pl.ds(base + 2688, 448)])
        plsc.subcore_barrier()


def _sc_seg16(tab4, src2d4, dst2d, ew1d):
    return pl.kernel(
        _sc_seg16_body,
        out_type=jax.ShapeDtypeStruct((4, NPAD, 16), jnp.float32),
        mesh=plsc.VectorSubcoreMesh(**_MESH),
        compiler_params=pltpu.CompilerParams(use_tc_tiling_on_sc=False),
        scratch_types=[
            pltpu.VMEM((_PHG, 128), jnp.int32),
            pltpu.VMEM((_PHG, 128), jnp.int32),
            pltpu.VMEM((_PHG * 128,), jnp.float32),
            pltpu.VMEM((_PHG, 128), jnp.int32),
            pltpu.VMEM((_PHG, 128), jnp.int32),
            pltpu.VMEM((_PHG * 128,), jnp.float32),
            pltpu.VMEM((_SUB * 128, 16), jnp.float32),
            pltpu.VMEM((_SUB * 128, 16), jnp.float32),
            pltpu.VMEM((_SUB * 128, 16), jnp.float32),
            pltpu.VMEM((_SUB * 128, 16), jnp.float32),
            pltpu.VMEM((64, 16), jnp.float32),
            pltpu.VMEM_SHARED((NPAD, 16), jnp.float32),
            pltpu.SemaphoreType.DMA,
            pltpu.SemaphoreType.DMA,
            pltpu.SemaphoreType.DMA,
            pltpu.SemaphoreType.DMA,
            pltpu.SemaphoreType.DMA,
            pltpu.SemaphoreType.DMA,
            pltpu.SemaphoreType.DMA,
        ],
    )(tab4, src2d4, dst2d, ew1d)


def _tc1_body(agg_ref, x_ref, wrel_ref, wroot_ref, b_ref, out_ref):
    a = agg_ref[0] + agg_ref[1]
    xv = x_ref[...]
    h = a * wrel_ref[...] + xv * wroot_ref[...] + b_ref[...]
    out_ref[...] = jnp.maximum(h, 0.0)


def _tc1(agg0, x_p, W_rel0, W_root0, b_rel0):
    return pl.pallas_call(
        _tc1_body,
        grid=(NBLKS,),
        in_specs=[
            pl.BlockSpec((2, BLK, 1), lambda i: (0, i, 0)),
            pl.BlockSpec((BLK, 1), lambda i: (i, 0)),
            pl.BlockSpec((1, HH), lambda i: (0, 0)),
            pl.BlockSpec((1, HH), lambda i: (0, 0)),
            pl.BlockSpec((1, HH), lambda i: (0, 0)),
        ],
        out_specs=pl.BlockSpec((BLK, HH), lambda i: (i, 0)),
        out_shape=jax.ShapeDtypeStruct((NPAD, HH), jnp.float32),
    )(agg0.reshape(2, NPAD, 1), x_p.reshape(NPAD, 1),
      W_rel0.reshape(1, HH), W_root0.reshape(1, HH), b_rel0.reshape(1, HH))


def _agg_dot(agg_ref, wrel_ref):
    acc = jnp.dot(agg_ref[0], wrel_ref[0:16, :],
                  preferred_element_type=jnp.float32)
    for qq in range(1, 4):
        acc += jnp.dot(agg_ref[qq], wrel_ref[16 * qq:16 * (qq + 1), :],
                       preferred_element_type=jnp.float32)
    return acc


def _tc_mid_body(agg_ref, hp_ref, wrel_ref, wroot_ref, b_ref, out_ref):
    h = (_agg_dot(agg_ref, wrel_ref)
         + jnp.dot(hp_ref[...], wroot_ref[...],
                   preferred_element_type=jnp.float32)
         + b_ref[...])
    out_ref[...] = jnp.maximum(h, 0.0)


def _tc_mid(agg, h_prev, W_rel, W_root, b_rel):
    return pl.pallas_call(
        _tc_mid_body,
        grid=(NBLKS,),
        in_specs=[
            pl.BlockSpec((4, BLK, 16), lambda i: (0, i, 0)),
            pl.BlockSpec((BLK, HH), lambda i: (i, 0)),
            pl.BlockSpec((HH, HH), lambda i: (0, 0)),
            pl.BlockSpec((HH, HH), lambda i: (0, 0)),
            pl.BlockSpec((1, HH), lambda i: (0, 0)),
        ],
        out_specs=pl.BlockSpec((BLK, HH), lambda i: (i, 0)),
        out_shape=jax.ShapeDtypeStruct((NPAD, HH), jnp.float32),
    )(agg, h_prev, W_rel, W_root, b_rel.reshape(1, HH))


def _tc_final_body(agg_ref, hp_ref, wrel_ref, wroot_ref, b_ref, wl_ref,
                   bl_ref, batch_ref, out_ref, sums, counts):
    i = pl.program_id(0)

    @pl.when(i == 0)
    def _():
        sums[...] = jnp.zeros_like(sums)
        counts[...] = jnp.zeros_like(counts)

    h = (_agg_dot(agg_ref, wrel_ref)
         + jnp.dot(hp_ref[...], wroot_ref[...],
                   preferred_element_type=jnp.float32)
         + b_ref[...])
    h = jnp.maximum(h, 0.0)
    s = jnp.sum(h * wl_ref[...], axis=1, keepdims=True)
    onehot = (batch_ref[...] ==
              lax.broadcasted_iota(jnp.int32, (BLK, GG), 1)
              ).astype(jnp.float32)
    sums[...] += jnp.sum(onehot * s, axis=0, keepdims=True)
    counts[...] += jnp.sum(onehot, axis=0, keepdims=True)

    @pl.when(i == NBLKS - 1)
    def _():
        out_ref[...] = sums[...] / jnp.maximum(counts[...], 1.0) + bl_ref[...]


def _tc_final(agg, h_prev, W_rel, W_root, b_rel, W_lin, b_lin, batch_p):
    return pl.pallas_call(
        _tc_final_body,
        grid=(NBLKS,),
        in_specs=[
            pl.BlockSpec((4, BLK, 16), lambda i: (0, i, 0)),
            pl.BlockSpec((BLK, HH), lambda i: (i, 0)),
            pl.BlockSpec((HH, HH), lambda i: (0, 0)),
            pl.BlockSpec((HH, HH), lambda i: (0, 0)),
            pl.BlockSpec((1, HH), lambda i: (0, 0)),
            pl.BlockSpec((1, HH), lambda i: (0, 0)),
            pl.BlockSpec((1, 1), lambda i: (0, 0)),
            pl.BlockSpec((BLK, 1), lambda i: (i, 0)),
        ],
        out_specs=pl.BlockSpec((1, GG), lambda i: (0, 0)),
        out_shape=jax.ShapeDtypeStruct((1, GG), jnp.float32),
        scratch_shapes=[
            pltpu.VMEM((1, GG), jnp.float32),
            pltpu.VMEM((1, GG), jnp.float32),
        ],
    )(agg, h_prev, W_rel, W_root, b_rel.reshape(1, HH),
      W_lin.reshape(1, HH), b_lin.reshape(1, 1), batch_p.reshape(NPAD, 1))


def kernel(x, edge_index, edge_attr, batch,
           W_rel0, b_rel0, W_root0,
           W_rel1, b_rel1, W_root1,
           W_rel2, b_rel2, W_root2,
           W_lin, b_lin):
    src = edge_index[0]
    dst = edge_index[1]
    ew = edge_attr.reshape(-1)

    pad_e = EPAD - EE
    zi = jnp.zeros((pad_e,), jnp.int32)
    src_p = jnp.concatenate([src, zi])
    src2d = src_p.reshape(NGROUPS, 128)
    src2d4 = (src_p * 4).reshape(NGROUPS, 128)
    dst2d = jnp.concatenate([dst, zi]).reshape(NGROUPS, 128)
    ew1d = jnp.concatenate([ew, jnp.zeros((pad_e,), jnp.float32)])

    x_p = jnp.concatenate([x.reshape(-1), jnp.zeros((NPAD - NN,),
                                                    jnp.float32)])
    batch_p = jnp.concatenate(
        [batch, jnp.full((NPAD - NN,), GG, jnp.int32)])

    agg0 = _sc_seg1(x_p, src2d, dst2d, ew1d)
    h0 = _tc1(agg0, x_p, W_rel0, W_root0, b_rel0)
    agg1 = _sc_seg16(h0.reshape(4 * NPAD, 16), src2d4, dst2d, ew1d)
    h1 = _tc_mid(agg1, h0, W_rel1, W_root1, b_rel1)
    agg2 = _sc_seg16(h1.reshape(4 * NPAD, 16), src2d4, dst2d, ew1d)
    out = _tc_final(agg2, h1, W_rel2, W_root2, b_rel2, W_lin, b_lin, batch_p)
    return out.reshape(GG)
